# Initial kernel scaffold; baseline (speedup 1.0000x reference)
import jax
import jax.numpy as jnp
from jax.experimental import pallas as pl


def kernel(x, edge_index, ppr_weight, Q_w, Q_b, W_w, W_b):
    raise NotImplementedError("write your pallas kernel here")


# speedup vs baseline: 1.6741x; 1.6741x over previous
import functools

import jax
import jax.numpy as jnp
from jax import lax
from jax.experimental import pallas as pl
from jax.experimental.pallas import tpu as pltpu
from jax.experimental.pallas import tpu_sc as plsc

N = 10000
F = 128
FE = 144
E = 320000
NC, NS = 2, 16
NW = NC * NS
CH = 128
EPW = 10112
EPAD = NW * EPW
NCH = EPW // CH
NPAD = 10240
RPT = NPAD // NS


def _q_body(x_ref, qwT_ref, qb_ref, o_ref):
    h = jnp.dot(x_ref[...], qwT_ref[...], preferred_element_type=jnp.float32)
    h = h + qb_ref[...]
    h = jnp.where(h >= 0, h, 0.01 * h)
    col = lax.broadcasted_iota(jnp.int32, (h.shape[0], FE - F), 1)
    pad = jnp.where(col == 0, 1.0, 0.0).astype(jnp.float32)
    o_ref[...] = jnp.concatenate([h, pad], axis=1)


def _transform_q(x, Q_w, Q_b):
    blk = 400
    grid = N // blk
    return pl.pallas_call(
        _q_body,
        grid=(grid,),
        in_specs=[
            pl.BlockSpec((blk, F), lambda i: (i, 0)),
            pl.BlockSpec((F, F), lambda i: (0, 0)),
            pl.BlockSpec((1, F), lambda i: (0, 0)),
        ],
        out_specs=pl.BlockSpec((blk, FE), lambda i: (i, 0)),
        out_shape=jax.ShapeDtypeStruct((N, FE), jnp.float32),
    )(x, Q_w.T, Q_b[None, :])


def _sc_body(src_hbm, dst_hbm, ppr_hbm, ht_hbm, z_hbm, out_hbm,
             src_v, dst_v, ppr_v, rows_v, hagg_sh, sem):
    c = lax.axis_index("c")
    s = lax.axis_index("s")
    wid = c * NS + s
    pltpu.sync_copy(z_hbm, hagg_sh.at[pl.ds(s * RPT, RPT)])
    plsc.subcore_barrier()

    iota = lax.iota(jnp.int32, 16)
    base = wid * EPW

    def chunk(j, carry):
        off = base + j * CH
        pltpu.sync_copy(src_hbm.at[pl.ds(off, CH)], src_v)
        pltpu.sync_copy(dst_hbm.at[pl.ds(off, CH)], dst_v)
        pltpu.sync_copy(ppr_hbm.at[pl.ds(off, CH)], ppr_v)
        pltpu.async_copy(ht_hbm.at[src_v], rows_v, sem).wait()

        pvs = [ppr_v[pl.ds(g * 16, 16)] for g in range(CH // 16)]
        idx0s = [iota + (g * 16) for g in range(CH // 16)]

        def col(cc, carry2):
            ccv = jnp.full((16,), 0, jnp.int32) + cc
            for g in range(CH // 16):
                vals = plsc.load_gather(rows_v, [idx0s[g], ccv])
                plsc.store_scatter(rows_v, [idx0s[g], ccv], vals * pvs[g])
            return carry2

        lax.fori_loop(0, FE, col, 0)
        pltpu.sync_copy(rows_v, hagg_sh.at[dst_v], add=True)
        return carry

    lax.fori_loop(0, NCH, chunk, 0)
    plsc.subcore_barrier()
    pltpu.sync_copy(hagg_sh.at[pl.ds(s * RPT, RPT)],
                    out_hbm.at[c, pl.ds(s * RPT, RPT)])


def _aggregate_sc(src, dst, ppr, ht, z):
    mesh = plsc.VectorSubcoreMesh(core_axis_name="c", subcore_axis_name="s")
    fn = pl.kernel(
        _sc_body,
        out_type=jax.ShapeDtypeStruct((NC, NPAD, FE), jnp.float32),
        mesh=mesh,
        scratch_types=[
            pltpu.VMEM((CH,), jnp.int32),
            pltpu.VMEM((CH,), jnp.int32),
            pltpu.VMEM((CH,), jnp.float32),
            pltpu.VMEM((CH, FE), jnp.float32),
            pltpu.VMEM_SHARED((NPAD, FE), jnp.float32),
            pltpu.SemaphoreType.DMA,
        ],
        compiler_params=pltpu.CompilerParams(use_tc_tiling_on_sc=False, needs_layout_passes=False),
    )
    return fn(src, dst, ppr, ht, z)


def _fin_body(x_ref, hp_ref, wxT_ref, whT_ref, wb_ref, o_ref):
    hp = hp_ref[0] + hp_ref[1]
    w = hp[:, F:F + 1]
    hagg = hp[:, :F] / jnp.where(w == 0, 1.0, w)
    acc = jnp.dot(x_ref[...], wxT_ref[...], preferred_element_type=jnp.float32)
    acc = acc + jnp.dot(hagg, whT_ref[...], preferred_element_type=jnp.float32)
    acc = acc + wb_ref[...]
    hnew = jnp.where(acc >= 0, acc, 0.01 * acc)
    nrm = jnp.sqrt(jnp.sum(hnew * hnew, axis=1, keepdims=True))
    o_ref[...] = hnew / jnp.where(nrm == 0, 1.0, nrm)


def _finalize(x, hp, W_w, W_b):
    blk = 400
    grid = N // blk
    return pl.pallas_call(
        _fin_body,
        grid=(grid,),
        in_specs=[
            pl.BlockSpec((blk, F), lambda i: (i, 0)),
            pl.BlockSpec((NC, blk, FE), lambda i: (0, i, 0)),
            pl.BlockSpec((F, F), lambda i: (0, 0)),
            pl.BlockSpec((F, F), lambda i: (0, 0)),
            pl.BlockSpec((1, F), lambda i: (0, 0)),
        ],
        out_specs=pl.BlockSpec((blk, F), lambda i: (i, 0)),
        out_shape=jax.ShapeDtypeStruct((N, F), jnp.float32),
    )(x, hp, W_w[:, :F].T, W_w[:, F:].T, W_b[None, :])


def kernel(x, edge_index, ppr_weight, Q_w, Q_b, W_w, W_b):
    src = edge_index[0].astype(jnp.int32)
    dst = edge_index[1].astype(jnp.int32)
    padn = EPAD - E
    src = jnp.pad(src, (0, padn))
    dst = jnp.pad(dst, (0, padn))
    ppr = jnp.pad(ppr_weight.astype(jnp.float32), (0, padn))
    ht = _transform_q(x, Q_w, Q_b)
    z = jnp.zeros((RPT, FE), jnp.float32)
    hp = _aggregate_sc(src, dst, ppr, ht, z)
    return _finalize(x, hp, W_w, W_b)
